# baseline (speedup 1.0000x reference)
import functools

import jax
import jax.numpy as jnp
from jax import lax
from jax.experimental import pallas as pl
from jax.experimental.pallas import tpu as pltpu
from jax.experimental.pallas import tpu_sc as plsc

_N_NEURON = 8192
_KV = 2048
_KP = 1024
_K = _KV + _KP
_B = 4096

_MBLK = 512

_NC = 2
_NS = 16
_NW = _NC * _NS
_RPG = 4
_GRP = _B // (_NW * _RPG)


def _embed_body(vis_ref, prp_ref, wv_ref, bv_ref, wp_ref, bp_ref, src_ref):
    sv = jnp.dot(vis_ref[...], wv_ref[...], preferred_element_type=jnp.float32)
    src_ref[:, :_KV] = jnp.maximum(sv + bv_ref[...], 0.0)
    sp = jnp.dot(prp_ref[...], wp_ref[...], preferred_element_type=jnp.float32)
    src_ref[:, _KV:] = jnp.maximum(sp + bp_ref[...], 0.0)


def _embed(vision, proprio, W_vision, b_vision, W_proprio, b_proprio):
    grid = (_B // _MBLK,)
    return pl.pallas_call(
        _embed_body,
        grid=grid,
        in_specs=[
            pl.BlockSpec((_MBLK, 1024), lambda i: (i, 0)),
            pl.BlockSpec((_MBLK, 512), lambda i: (i, 0)),
            pl.BlockSpec((1024, _KV), lambda i: (0, 0)),
            pl.BlockSpec((1, _KV), lambda i: (0, 0)),
            pl.BlockSpec((512, _KP), lambda i: (0, 0)),
            pl.BlockSpec((1, _KP), lambda i: (0, 0)),
        ],
        out_specs=pl.BlockSpec((_MBLK, _K), lambda i: (i, 0)),
        out_shape=jax.ShapeDtypeStruct((_B, _K), jnp.float32),
    )(vision, proprio, W_vision, b_vision.reshape(1, _KV),
      W_proprio, b_proprio.reshape(1, _KP))


@functools.partial(
    pl.kernel,
    out_type=jax.ShapeDtypeStruct((_B * _N_NEURON,), jnp.float32),
    mesh=plsc.VectorSubcoreMesh(core_axis_name="c", subcore_axis_name="s"),
    scratch_types=[
        pltpu.VMEM((_K,), jnp.int32),
        pltpu.VMEM((_RPG * _K,), jnp.int32),
        pltpu.VMEM((_RPG * _K,), jnp.float32),
        pltpu.VMEM((_RPG * _N_NEURON,), jnp.float32),
        pltpu.VMEM_SHARED((_NS * _RPG * _N_NEURON,), jnp.float32),
    ],
)
def _sc_scatter(src_hbm, ids_hbm, out_hbm, ids_v, idx_v, src_v, zero_v, acc):
    s = lax.axis_index("s")
    c = lax.axis_index("c")
    w = c * _NS + s

    pltpu.sync_copy(ids_hbm, ids_v)

    zeros16 = jnp.zeros((16,), jnp.float32)

    def zloop(i, _):
        zero_v[pl.ds(i * 16, 16)] = zeros16
        return 0

    lax.fori_loop(0, _RPG * _N_NEURON // 16, zloop, 0, unroll=8)

    def iloop(i, _):
        l = i // (_K // 16)
        j = i % (_K // 16)
        idx_v[pl.ds(i * 16, 16)] = (
            ids_v[pl.ds(j * 16, 16)] + (s * _RPG + l) * _N_NEURON
        )
        return 0

    lax.fori_loop(0, _RPG * _K // 16, iloop, 0, unroll=4)

    base = s * _RPG * _N_NEURON

    def gloop(g, _):
        row0 = (w * _GRP + g) * _RPG
        pltpu.sync_copy(zero_v, acc.at[pl.ds(base, _RPG * _N_NEURON)])
        pltpu.sync_copy(src_hbm.at[pl.ds(row0 * _K, _RPG * _K)], src_v)
        pltpu.sync_copy(src_v, acc.at[idx_v], add=True)
        pltpu.sync_copy(
            acc.at[pl.ds(base, _RPG * _N_NEURON)],
            out_hbm.at[pl.ds(row0 * _N_NEURON, _RPG * _N_NEURON)],
        )
        return 0

    lax.fori_loop(0, _GRP, gloop, 0)


def kernel(vision, proprio, W_vision, b_vision, W_proprio, b_proprio,
           ids_vision, ids_proprio):
    src = _embed(vision, proprio, W_vision, b_vision, W_proprio, b_proprio)
    ids = jnp.concatenate([ids_vision, ids_proprio])
    out = _sc_scatter(src.reshape(_B * _K), ids)
    return out.reshape(_B, _N_NEURON)


# speedup vs baseline: 1.1754x; 1.1754x over previous
import functools

import jax
import jax.numpy as jnp
from jax import lax
from jax.experimental import pallas as pl
from jax.experimental.pallas import tpu as pltpu
from jax.experimental.pallas import tpu_sc as plsc

_N_NEURON = 8192
_KV = 2048
_KP = 1024
_K = _KV + _KP
_B = 4096

_MBLK = 512

_NC = 2
_NS = 16
_NW = _NC * _NS
_RPG = 4
_GRP = _B // (_NW * _RPG)


def _embed_body(vis_ref, prp_ref, wv_ref, bv_ref, wp_ref, bp_ref, src_ref):
    sv = jnp.dot(vis_ref[...], wv_ref[...], preferred_element_type=jnp.float32)
    src_ref[:, :_KV] = jnp.maximum(sv + bv_ref[...], 0.0)
    sp = jnp.dot(prp_ref[...], wp_ref[...], preferred_element_type=jnp.float32)
    src_ref[:, _KV:] = jnp.maximum(sp + bp_ref[...], 0.0)


def _embed(vision, proprio, W_vision, b_vision, W_proprio, b_proprio):
    grid = (_B // _MBLK,)
    return pl.pallas_call(
        _embed_body,
        grid=grid,
        in_specs=[
            pl.BlockSpec((_MBLK, 1024), lambda i: (i, 0)),
            pl.BlockSpec((_MBLK, 512), lambda i: (i, 0)),
            pl.BlockSpec((1024, _KV), lambda i: (0, 0)),
            pl.BlockSpec((1, _KV), lambda i: (0, 0)),
            pl.BlockSpec((512, _KP), lambda i: (0, 0)),
            pl.BlockSpec((1, _KP), lambda i: (0, 0)),
        ],
        out_specs=pl.BlockSpec((_MBLK, _K), lambda i: (i, 0)),
        out_shape=jax.ShapeDtypeStruct((_B, _K), jnp.float32),
    )(vision, proprio, W_vision, b_vision.reshape(1, _KV),
      W_proprio, b_proprio.reshape(1, _KP))


@functools.partial(
    pl.kernel,
    out_type=jax.ShapeDtypeStruct((_B, _N_NEURON), jnp.float32),
    mesh=plsc.VectorSubcoreMesh(core_axis_name="c", subcore_axis_name="s"),
    scratch_types=[
        pltpu.VMEM((_K,), jnp.int32),
        pltpu.VMEM((_RPG * _K,), jnp.int32),
        pltpu.VMEM((_RPG * _K,), jnp.float32),
        pltpu.VMEM((_RPG * _N_NEURON,), jnp.float32),
        pltpu.VMEM_SHARED((_NS * _RPG * _N_NEURON,), jnp.float32),
    ],
)
def _sc_scatter(src_hbm, ids_hbm, out_hbm, ids_v, idx_v, src_v, zero_v, acc):
    s = lax.axis_index("s")
    c = lax.axis_index("c")
    w = c * _NS + s

    pltpu.sync_copy(ids_hbm, ids_v)

    zeros16 = jnp.zeros((16,), jnp.float32)

    def zloop(i, _):
        zero_v[pl.ds(i * 16, 16)] = zeros16
        return 0

    lax.fori_loop(0, _RPG * _N_NEURON // 16, zloop, 0, unroll=8)

    def iloop(i, _):
        l = i // (_K // 16)
        j = i % (_K // 16)
        idx_v[pl.ds(i * 16, 16)] = (
            ids_v[pl.ds(j * 16, 16)] + (s * _RPG + l) * _N_NEURON
        )
        return 0

    lax.fori_loop(0, _RPG * _K // 16, iloop, 0, unroll=4)

    base = s * _RPG * _N_NEURON

    def gloop(g, _):
        row0 = (w * _GRP + g) * _RPG
        pltpu.sync_copy(zero_v, acc.at[pl.ds(base, _RPG * _N_NEURON)])

        def srcloop(l, _):
            pltpu.sync_copy(
                src_hbm.at[row0 + l], src_v.at[pl.ds(l * _K, _K)]
            )
            return 0

        lax.fori_loop(0, _RPG, srcloop, 0, unroll=True)
        pltpu.sync_copy(src_v, acc.at[idx_v], add=True)

        def outloop(l, _):
            pltpu.sync_copy(
                acc.at[pl.ds(base + l * _N_NEURON, _N_NEURON)],
                out_hbm.at[row0 + l],
            )
            return 0

        lax.fori_loop(0, _RPG, outloop, 0, unroll=True)
        return 0

    lax.fori_loop(0, _GRP, gloop, 0)


def kernel(vision, proprio, W_vision, b_vision, W_proprio, b_proprio,
           ids_vision, ids_proprio):
    src = _embed(vision, proprio, W_vision, b_vision, W_proprio, b_proprio)
    ids = jnp.concatenate([ids_vision, ids_proprio])
    return _sc_scatter(src, ids)
